# baseline (speedup 1.0000x reference)
import functools

import jax
import jax.numpy as jnp
from jax import lax
from jax.experimental import pallas as pl
from jax.experimental.pallas import tpu as pltpu
from jax.experimental.pallas import tpu_sc as plsc


@functools.partial(jax.jit, static_argnums=(2, 3, 4, 5))
def _gather_rows(table, idx, B, P, S, D):
    info = plsc.get_sparse_core_info()
    NC, NS, L = info.num_cores, info.num_subcores, info.num_lanes
    NW = NC * NS
    N = B * P
    assert N % NW == 0
    b_per_w = N // NW
    assert b_per_w % L == 0 and (b_per_w * D * 4) <= 500_000

    mesh = plsc.VectorSubcoreMesh(core_axis_name="c", subcore_axis_name="s")

    C = 16
    n_chunks = b_per_w // C

    @functools.partial(
        pl.kernel,
        mesh=mesh,
        out_type=jax.ShapeDtypeStruct((N, D), jnp.float32),
        scratch_types=[
            pltpu.VMEM((b_per_w,), jnp.int32),
            pltpu.VMEM((2, C, D), jnp.float32),
            pltpu.SemaphoreType.DMA,
            pltpu.SemaphoreType.DMA,
            pltpu.SemaphoreType.DMA,
            pltpu.SemaphoreType.DMA,
        ],
    )
    def k(table_hbm, idx_hbm, out_hbm, idx_v, rows_v, g0, g1, s0, s1):
        wid = lax.axis_index("s") * NC + lax.axis_index("c")
        base = wid * b_per_w
        pltpu.sync_copy(idx_hbm.at[pl.ds(base, b_per_w)], idx_v)
        off = (base // P) * S
        for i in range(b_per_w // L):
            idx_v[pl.ds(i * L, L)] = idx_v[pl.ds(i * L, L)] + off
        gsem = (g0, g1)
        ssem = (s0, s1)
        gathers = [
            pltpu.make_async_copy(
                table_hbm.at[idx_v.at[pl.ds(j * C, C)]],
                rows_v.at[j % 2],
                gsem[j % 2],
            )
            for j in range(n_chunks)
        ]
        stores = [
            pltpu.make_async_copy(
                rows_v.at[j % 2],
                out_hbm.at[pl.ds(base + j * C, C)],
                ssem[j % 2],
            )
            for j in range(n_chunks)
        ]
        gathers[0].start()
        for j in range(n_chunks):
            gathers[j].wait()
            if j + 1 < n_chunks:
                if j + 1 >= 2:
                    stores[j - 1].wait()
                gathers[j + 1].start()
            stores[j].start()
        if n_chunks >= 2:
            stores[n_chunks - 2].wait()
        stores[n_chunks - 1].wait()

    return k(table, idx)


def kernel(sequence_tensor, masked_lm_positions):
    B, S, D = sequence_tensor.shape
    _, P = masked_lm_positions.shape
    table = sequence_tensor.reshape(B * S, D)
    idx = masked_lm_positions.astype(jnp.int32).reshape(B * P)
    out = _gather_rows(table, idx, B, P, S, D)
    return out.reshape(B, P, D)


# speedup vs baseline: 1.0248x; 1.0248x over previous
import functools

import jax
import jax.numpy as jnp
from jax import lax
from jax.experimental import pallas as pl
from jax.experimental.pallas import tpu as pltpu
from jax.experimental.pallas import tpu_sc as plsc


@functools.partial(jax.jit, static_argnums=(2, 3, 4, 5))
def _gather_rows(table, idx, B, P, S, D):
    info = plsc.get_sparse_core_info()
    NC, NS, L = info.num_cores, info.num_subcores, info.num_lanes
    NW = NC * NS
    N = B * P
    assert N % NW == 0
    b_per_w = N // NW
    assert b_per_w % L == 0 and (b_per_w * D * 4) <= 500_000

    mesh = plsc.VectorSubcoreMesh(core_axis_name="c", subcore_axis_name="s")

    C = b_per_w // 2

    @functools.partial(
        pl.kernel,
        mesh=mesh,
        out_type=jax.ShapeDtypeStruct((N, D), jnp.float32),
        scratch_types=[
            pltpu.VMEM((2, C), jnp.int32),
            pltpu.VMEM((2, C, D), jnp.float32),
            pltpu.SemaphoreType.DMA,
            pltpu.SemaphoreType.DMA,
            pltpu.SemaphoreType.DMA,
            pltpu.SemaphoreType.DMA,
        ],
    )
    def k(table_hbm, idx_hbm, out_hbm, idx_v, rows_v, g0, g1, s0, s1):
        wid = lax.axis_index("s") * NC + lax.axis_index("c")
        base = wid * b_per_w
        for h in range(2):
            pltpu.sync_copy(idx_hbm.at[pl.ds(base + h * C, C)], idx_v.at[h])
        off = (base // P) * S
        for h in range(2):
            for i in range(C // L):
                idx_v[h, pl.ds(i * L, L)] = idx_v[h, pl.ds(i * L, L)] + off
        gsem = (g0, g1)
        ssem = (s0, s1)
        gathers = [
            pltpu.make_async_copy(
                table_hbm.at[idx_v.at[h]], rows_v.at[h], gsem[h]
            )
            for h in range(2)
        ]
        stores = [
            pltpu.make_async_copy(
                rows_v.at[h], out_hbm.at[pl.ds(base + h * C, C)], ssem[h]
            )
            for h in range(2)
        ]
        gathers[0].start()
        gathers[0].wait()
        gathers[1].start()
        stores[0].start()
        gathers[1].wait()
        stores[1].start()
        stores[0].wait()
        stores[1].wait()

    return k(table, idx)


def kernel(sequence_tensor, masked_lm_positions):
    B, S, D = sequence_tensor.shape
    _, P = masked_lm_positions.shape
    table = sequence_tensor.reshape(B * S, D)
    idx = masked_lm_positions.astype(jnp.int32).reshape(B * P)
    out = _gather_rows(table, idx, B, P, S, D)
    return out.reshape(B, P, D)


# speedup vs baseline: 1.0830x; 1.0568x over previous
import functools

import jax
import jax.numpy as jnp
from jax import lax
from jax.experimental import pallas as pl
from jax.experimental.pallas import tpu as pltpu
from jax.experimental.pallas import tpu_sc as plsc


@functools.partial(jax.jit, static_argnums=(2, 3, 4, 5))
def _gather_rows(table, idx, B, P, S, D):
    info = plsc.get_sparse_core_info()
    NC, NS, L = info.num_cores, info.num_subcores, info.num_lanes
    NW = NC * NS
    N = B * P
    assert N % NW == 0
    b_per_w = N // NW
    assert b_per_w % L == 0 and (b_per_w * D * 4) <= 500_000

    mesh = plsc.VectorSubcoreMesh(core_axis_name="c", subcore_axis_name="s")

    @functools.partial(
        pl.kernel,
        mesh=mesh,
        out_type=jax.ShapeDtypeStruct((N, D), jnp.float32),
        scratch_types=[
            pltpu.VMEM((b_per_w,), jnp.int32),
            pltpu.VMEM((b_per_w, D), jnp.float32),
            pltpu.SemaphoreType.DMA,
        ],
    )
    def k(table_hbm, idx_hbm, out_hbm, idx_v, rows_v, sem):
        wid = lax.axis_index("s") * NC + lax.axis_index("c")
        base = wid * b_per_w
        b = base // P
        col = base - b * P
        pltpu.sync_copy(idx_hbm.at[b, pl.ds(col, b_per_w)], idx_v)
        off = b * S
        for i in range(b_per_w // L):
            idx_v[pl.ds(i * L, L)] = idx_v[pl.ds(i * L, L)] + off
        pltpu.async_copy(table_hbm.at[idx_v], rows_v, sem).wait()
        pltpu.sync_copy(rows_v, out_hbm.at[pl.ds(base, b_per_w)])

    return k(table, idx)


def kernel(sequence_tensor, masked_lm_positions):
    B, S, D = sequence_tensor.shape
    _, P = masked_lm_positions.shape
    table = sequence_tensor.reshape(B * S, D)
    idx = masked_lm_positions.astype(jnp.int32)
    out = _gather_rows(table, idx, B, P, S, D)
    return out.reshape(B, P, D)
